# baseline (speedup 1.0000x reference)
import functools

import jax
import jax.numpy as jnp
from jax import lax
from jax.experimental import pallas as pl
from jax.experimental.pallas import tpu as pltpu
from jax.experimental.pallas import tpu_sc as plsc

B, S, H = 4, 2048, 768
D, U = 50, 32
N = B * S
INNER = 4 * H

NC, NS = 2, 16
NW = NC * NS

WG = 64
XCH = 128
XG = 5 * XCH


def _sc_words_body(wtab, wids, dep, words_out, widx_v, wbuf, sem_w):
    del dep
    c = lax.axis_index("c")
    s = lax.axis_index("s")
    wid = s * NC + c
    tpw = wids.shape[0] // NW
    wbase = wid * tpw

    def wloop(g, carry):
        r0 = wbase + g * WG
        pltpu.sync_copy(wids.at[pl.ds(r0, WG)], widx_v)
        pltpu.async_copy(wtab.at[widx_v], wbuf, sem_w).wait()
        pltpu.sync_copy(wbuf, words_out.at[pl.ds(r0, WG)])
        return carry

    lax.fori_loop(0, tpw // WG, wloop, 0)


def _sc_words(word_emb, wids, dep):
    mesh = plsc.VectorSubcoreMesh(core_axis_name="c", subcore_axis_name="s")
    f = pl.kernel(
        _sc_words_body,
        out_type=jax.ShapeDtypeStruct((wids.shape[0], H), jnp.float32),
        mesh=mesh,
        scratch_types=[
            pltpu.VMEM((WG,), jnp.int32),
            pltpu.VMEM((WG, H), jnp.float32),
            pltpu.SemaphoreType.DMA,
        ],
    )
    return f(word_emb, wids, dep)


def _sc_xpath_body(tag_t, tag_i, sub_t, sub_i, dep, xp_out,
                   tix_v, six_v, tbuf, sbuf, sem_t, sem_s):
    del dep
    c = lax.axis_index("c")
    s = lax.axis_index("s")
    wid = s * NC + c
    ipw = tag_i.shape[0] // NW
    xbase = wid * ipw

    def xloop(g, carry):
        o0 = xbase + g * XG
        pltpu.sync_copy(tag_i.at[pl.ds(o0, XG)], tix_v)
        pltpu.sync_copy(sub_i.at[pl.ds(o0, XG)], six_v)
        handles = []
        for k in range(XG // XCH):
            sl = pl.ds(k * XCH, XCH)
            handles.append(pltpu.async_copy(tag_t.at[tix_v.at[sl]], tbuf.at[sl], sem_t))
            handles.append(pltpu.async_copy(sub_t.at[six_v.at[sl]], sbuf.at[sl], sem_s))
        for h in handles:
            h.wait()

        @plsc.parallel_loop(0, XG, step=1, unroll=8)
        def add_body(i):
            tbuf[i, pl.ds(0, 16)] = tbuf[i, pl.ds(0, 16)] + sbuf[i, pl.ds(0, 16)]
            tbuf[i, pl.ds(16, 16)] = tbuf[i, pl.ds(16, 16)] + sbuf[i, pl.ds(16, 16)]

        pltpu.sync_copy(tbuf, xp_out.at[pl.ds(o0, XG)])
        return carry

    lax.fori_loop(0, ipw // XG, xloop, 0)


def _sc_xpath(tag_t, tag_i, sub_t, sub_i, dep):
    mesh = plsc.VectorSubcoreMesh(core_axis_name="c", subcore_axis_name="s")
    f = pl.kernel(
        _sc_xpath_body,
        out_type=jax.ShapeDtypeStruct((tag_i.shape[0], U), jnp.float32),
        mesh=mesh,
        scratch_types=[
            pltpu.VMEM((XG,), jnp.int32),
            pltpu.VMEM((XG,), jnp.int32),
            pltpu.VMEM((XG, U), jnp.float32),
            pltpu.VMEM((XG, U), jnp.float32),
            pltpu.SemaphoreType.DMA,
            pltpu.SemaphoreType.DMA,
        ],
        compiler_params=pltpu.CompilerParams(use_tc_tiling_on_sc=False),
    )
    return f(tag_t, tag_i, sub_t, sub_i, dep)


def _tc_body(xp, words, pos, te, wi, bi, wie, bie, w1, b1, w2, b2,
             g, bb, out):
    x = xp[...]
    h = jnp.dot(x, wi[...], preferred_element_type=jnp.float32) + bi[...]
    h = jnp.maximum(h, 0.0).astype(jnp.bfloat16)
    xe = jnp.dot(h, wie[...], preferred_element_type=jnp.float32) + bie[...]
    t = jnp.maximum(jnp.dot(xe.astype(jnp.bfloat16), w1[...],
                            preferred_element_type=jnp.float32) + b1[...], 0.0)
    t = jnp.dot(t.astype(jnp.bfloat16), w2[...],
                preferred_element_type=jnp.float32) + b2[...]
    e = words[...] + pos[...] + te[0:1, :] + t
    m = jnp.mean(e, axis=-1, keepdims=True)
    v = jnp.mean((e - m) ** 2, axis=-1, keepdims=True)
    out[...] = (e - m) / jnp.sqrt(v + 1e-12) * g[...] + bb[...]


def _tc_fused(xp, words, pos_emb, type_emb, wi, bi, wie, bie,
              w1, b1, w2, b2, g, bb):
    TB = 512
    ntok = xp.shape[0]
    grid = (ntok // TB,)
    im_tok = lambda i: (i, 0)
    im_pos = lambda i: (i % (S // TB), 0)
    im0 = lambda i: (0, 0)
    return pl.pallas_call(
        _tc_body,
        grid=grid,
        in_specs=[
            pl.BlockSpec((TB, D * U), im_tok),
            pl.BlockSpec((TB, H), im_tok),
            pl.BlockSpec((TB, H), im_pos),
            pl.BlockSpec((2, H), im0),
            pl.BlockSpec((D * U, INNER), im0),
            pl.BlockSpec((1, INNER), im0),
            pl.BlockSpec((INNER, H), im0),
            pl.BlockSpec((1, H), im0),
            pl.BlockSpec((H, H), im0),
            pl.BlockSpec((1, H), im0),
            pl.BlockSpec((H, H), im0),
            pl.BlockSpec((1, H), im0),
            pl.BlockSpec((1, H), im0),
            pl.BlockSpec((1, H), im0),
        ],
        out_specs=pl.BlockSpec((TB, H), im_tok),
        out_shape=jax.ShapeDtypeStruct((ntok, H), jnp.float32),
        compiler_params=pltpu.CompilerParams(
            dimension_semantics=("arbitrary",),
        ),
    )(xp, words, pos_emb, type_emb, wi, bi, wie, bie,
      w1, b1, w2, b2, g, bb)


def kernel(input_ids, xpath_tags_seq, xpath_subs_seq, word_emb, pos_emb,
           type_emb, tag_tables, subs_tables, W_inner, b_inner, W_i2e, b_i2e,
           W_wl1, b_wl1, W_wl2, b_wl2, ln_g, ln_b):
    tagv = tag_tables.shape[1]
    subv = subs_tables.shape[1]
    wids = input_ids.reshape(-1).astype(jnp.int32)
    tag_i = (xpath_tags_seq.astype(jnp.int32)
             + jnp.arange(D, dtype=jnp.int32) * tagv).reshape(-1)
    sub_i = (xpath_subs_seq.astype(jnp.int32)
             + jnp.arange(D, dtype=jnp.int32) * subv).reshape(-1)
    tag_t = tag_tables.reshape(D * tagv, U)
    sub_t = subs_tables.reshape(D * subv, U)

    wi = W_inner.astype(jnp.bfloat16)
    wie = W_i2e.astype(jnp.bfloat16)
    w1_ = W_wl1.astype(jnp.bfloat16)
    w2 = W_wl2.astype(jnp.bfloat16)
    bi = b_inner.reshape(1, INNER)
    bie = b_i2e.reshape(1, H)
    b1 = b_wl1.reshape(1, H)
    b2 = b_wl2.reshape(1, H)
    g = ln_g.reshape(1, H)
    bb = ln_b.reshape(1, H)

    NH = N // 2
    w0 = _sc_words(word_emb, wids[:NH], type_emb[:1])
    xp0 = _sc_xpath(tag_t, tag_i[:NH * D], sub_t, sub_i[:NH * D],
                    w0[:8])
    w1 = _sc_words(word_emb, wids[NH:], xp0[:1])
    xp1 = _sc_xpath(tag_t, tag_i[NH * D:], sub_t, sub_i[NH * D:],
                    w1[:8])
    out0 = _tc_fused(xp0.reshape(NH, D * U), w0, pos_emb,
                     type_emb, wi, bi, wie, bie, w1_, b1, w2, b2, g, bb)
    out1 = _tc_fused(xp1.reshape(NH, D * U), w1, pos_emb,
                     type_emb, wi, bi, wie, bie, w1_, b1, w2, b2, g, bb)
    out = jnp.concatenate([out0, out1], axis=0)
    return out.reshape(B, S, H)


# speedup vs baseline: 1.3032x; 1.3032x over previous
import functools

import jax
import jax.numpy as jnp
from jax import lax
from jax.experimental import pallas as pl
from jax.experimental.pallas import tpu as pltpu
from jax.experimental.pallas import tpu_sc as plsc

B, S, H = 4, 2048, 768
D, U = 50, 32
N = B * S
INNER = 4 * H

NC, NS = 2, 16
NW = NC * NS

WG = 64
XCH = 128
XG = 5 * XCH


def _sc_words_body(wtab, wids, words_out, widx_v, wbuf, sem_w, sem_o):
    c = lax.axis_index("c")
    s = lax.axis_index("s")
    wid = s * NC + c
    tpw = wids.shape[0] // NW
    wbase = wid * tpw
    ng = tpw // WG

    def li(g, slot):
        pltpu.sync_copy(wids.at[pl.ds(wbase + g * WG, WG)], widx_v.at[slot])

    def fire(slot):
        return pltpu.async_copy(wtab.at[widx_v.at[slot]], wbuf.at[slot], sem_w)

    def put(g, slot):
        return pltpu.async_copy(wbuf.at[slot],
                                words_out.at[pl.ds(wbase + g * WG, WG)], sem_o)

    li(0, 0)
    h = [fire(0), None]
    li(1, 1)
    h[1] = fire(1)
    tail = []
    for g in range(ng):
        slot = g % 2
        h[slot].wait()
        p = put(g, slot)
        if g + 2 < ng:
            li(g + 2, slot)
            p.wait()
            h[slot] = fire(slot)
        else:
            tail.append(p)
    for p in tail:
        p.wait()


def _sc_words(word_emb, wids):
    mesh = plsc.VectorSubcoreMesh(core_axis_name="c", subcore_axis_name="s")
    f = pl.kernel(
        _sc_words_body,
        out_type=jax.ShapeDtypeStruct((wids.shape[0], H), jnp.float32),
        mesh=mesh,
        scratch_types=[
            pltpu.VMEM((2, WG), jnp.int32),
            pltpu.VMEM((2, WG, H), jnp.float32),
            pltpu.SemaphoreType.DMA,
            pltpu.SemaphoreType.DMA,
        ],
    )
    return f(word_emb, wids)


def _sc_xpath_body(tag_t, tag_i, sub_t, sub_i, xp_out,
                   tix, six, tbuf, sbuf, sem_t, sem_s, sem_o):
    c = lax.axis_index("c")
    s = lax.axis_index("s")
    wid = s * NC + c
    ipw = tag_i.shape[0] // NW
    xbase = wid * ipw
    ng = ipw // XG
    npair = ng // 2

    def li(g, slot):
        o0 = xbase + g * XG
        pltpu.sync_copy(tag_i.at[pl.ds(o0, XG)], tix.at[slot])
        pltpu.sync_copy(sub_i.at[pl.ds(o0, XG)], six.at[slot])

    def fire(slot):
        for k in range(XG // XCH):
            sl = pl.ds(k * XCH, XCH)
            pltpu.async_copy(tag_t.at[tix.at[slot, sl]], tbuf.at[slot, sl], sem_t)
            pltpu.async_copy(sub_t.at[six.at[slot, sl]], sbuf.at[slot, sl], sem_s)

    def drain_gathers(slot):
        for k in range(XG // XCH):
            sl = pl.ds(k * XCH, XCH)
            pltpu.make_async_copy(tag_t.at[tix.at[slot, sl]], tbuf.at[slot, sl], sem_t).wait()
            pltpu.make_async_copy(sub_t.at[six.at[slot, sl]], sbuf.at[slot, sl], sem_s).wait()

    def adds(slot):
        @plsc.parallel_loop(0, XG, step=1, unroll=8)
        def add_body(i):
            tbuf[slot, i, pl.ds(0, 16)] = (tbuf[slot, i, pl.ds(0, 16)]
                                           + sbuf[slot, i, pl.ds(0, 16)])
            tbuf[slot, i, pl.ds(16, 16)] = (tbuf[slot, i, pl.ds(16, 16)]
                                            + sbuf[slot, i, pl.ds(16, 16)])

    def put(g, slot):
        pltpu.async_copy(tbuf.at[slot], xp_out.at[pl.ds(xbase + g * XG, XG)], sem_o)

    def drain_put(g, slot):
        pltpu.make_async_copy(tbuf.at[slot],
                              xp_out.at[pl.ds(xbase + g * XG, XG)], sem_o).wait()

    li(0, 0)
    fire(0)
    li(1, 1)
    fire(1)

    def pair(p, carry):
        g0 = 2 * p
        for slot in range(2):
            g = g0 + slot
            drain_gathers(slot)
            adds(slot)
            put(g, slot)
            li(g + 2, slot)
            drain_put(g, slot)
            fire(slot)
        return carry

    lax.fori_loop(0, npair - 1, pair, 0)

    for slot in range(2):
        g = ng - 2 + slot
        drain_gathers(slot)
        adds(slot)
        put(g, slot)
    for slot in range(2):
        drain_put(ng - 2 + slot, slot)


def _sc_xpath(tag_t, tag_i, sub_t, sub_i):
    mesh = plsc.VectorSubcoreMesh(core_axis_name="c", subcore_axis_name="s")
    f = pl.kernel(
        _sc_xpath_body,
        out_type=jax.ShapeDtypeStruct((tag_i.shape[0], U), jnp.float32),
        mesh=mesh,
        scratch_types=[
            pltpu.VMEM((2, XG), jnp.int32),
            pltpu.VMEM((2, XG), jnp.int32),
            pltpu.VMEM((2, XG, U), jnp.float32),
            pltpu.VMEM((2, XG, U), jnp.float32),
            pltpu.SemaphoreType.DMA,
            pltpu.SemaphoreType.DMA,
            pltpu.SemaphoreType.DMA,
        ],
        compiler_params=pltpu.CompilerParams(use_tc_tiling_on_sc=False),
    )
    return f(tag_t, tag_i, sub_t, sub_i)


def _tc_body(xp, words, pos, te, wi, bi, wie, bie, w1, b1, w2, b2,
             g, bb, out):
    x = xp[...]
    h = jnp.dot(x, wi[...], preferred_element_type=jnp.float32) + bi[...]
    h = jnp.maximum(h, 0.0).astype(jnp.bfloat16)
    xe = jnp.dot(h, wie[...], preferred_element_type=jnp.float32) + bie[...]
    t = jnp.maximum(jnp.dot(xe.astype(jnp.bfloat16), w1[...],
                            preferred_element_type=jnp.float32) + b1[...], 0.0)
    t = jnp.dot(t.astype(jnp.bfloat16), w2[...],
                preferred_element_type=jnp.float32) + b2[...]
    e = words[...] + pos[...] + te[0:1, :] + t
    m = jnp.mean(e, axis=-1, keepdims=True)
    v = jnp.mean((e - m) ** 2, axis=-1, keepdims=True)
    out[...] = (e - m) / jnp.sqrt(v + 1e-12) * g[...] + bb[...]


def _tc_fused(xp, words, pos_emb, type_emb, wi, bi, wie, bie,
              w1, b1, w2, b2, g, bb):
    TB = 512
    ntok = xp.shape[0]
    grid = (ntok // TB,)
    im_tok = lambda i: (i, 0)
    im_pos = lambda i: (i % (S // TB), 0)
    im0 = lambda i: (0, 0)
    return pl.pallas_call(
        _tc_body,
        grid=grid,
        in_specs=[
            pl.BlockSpec((TB, D * U), im_tok),
            pl.BlockSpec((TB, H), im_tok),
            pl.BlockSpec((TB, H), im_pos),
            pl.BlockSpec((2, H), im0),
            pl.BlockSpec((D * U, INNER), im0),
            pl.BlockSpec((1, INNER), im0),
            pl.BlockSpec((INNER, H), im0),
            pl.BlockSpec((1, H), im0),
            pl.BlockSpec((H, H), im0),
            pl.BlockSpec((1, H), im0),
            pl.BlockSpec((H, H), im0),
            pl.BlockSpec((1, H), im0),
            pl.BlockSpec((1, H), im0),
            pl.BlockSpec((1, H), im0),
        ],
        out_specs=pl.BlockSpec((TB, H), im_tok),
        out_shape=jax.ShapeDtypeStruct((ntok, H), jnp.float32),
        compiler_params=pltpu.CompilerParams(
            dimension_semantics=("arbitrary",),
        ),
    )(xp, words, pos_emb, type_emb, wi, bi, wie, bie,
      w1, b1, w2, b2, g, bb)


def kernel(input_ids, xpath_tags_seq, xpath_subs_seq, word_emb, pos_emb,
           type_emb, tag_tables, subs_tables, W_inner, b_inner, W_i2e, b_i2e,
           W_wl1, b_wl1, W_wl2, b_wl2, ln_g, ln_b):
    tagv = tag_tables.shape[1]
    subv = subs_tables.shape[1]
    wids = input_ids.reshape(-1).astype(jnp.int32)
    tag_i = (xpath_tags_seq.astype(jnp.int32)
             + jnp.arange(D, dtype=jnp.int32) * tagv).reshape(-1)
    sub_i = (xpath_subs_seq.astype(jnp.int32)
             + jnp.arange(D, dtype=jnp.int32) * subv).reshape(-1)
    tag_t = tag_tables.reshape(D * tagv, U)
    sub_t = subs_tables.reshape(D * subv, U)

    words = _sc_words(word_emb, wids)
    xp = _sc_xpath(tag_t, tag_i, sub_t, sub_i)

    out = _tc_fused(
        xp.reshape(N, D * U), words, pos_emb,
        type_emb, W_inner.astype(jnp.bfloat16), b_inner.reshape(1, INNER),
        W_i2e.astype(jnp.bfloat16), b_i2e.reshape(1, H),
        W_wl1.astype(jnp.bfloat16), b_wl1.reshape(1, H),
        W_wl2.astype(jnp.bfloat16), b_wl2.reshape(1, H),
        ln_g.reshape(1, H), ln_b.reshape(1, H))
    return out.reshape(B, S, H)
